# baseline (speedup 1.0000x reference)
import functools

import jax
import jax.numpy as jnp
from jax.experimental import pallas as pl
from jax.experimental.pallas import tpu as pltpu
from jax.experimental.pallas import tpu_sc as plsc


def _conv(x, w, b, stride, pad):
    y = jax.lax.conv_general_dilated(
        x, w, (stride, stride), ((pad, pad), (pad, pad)),
        dimension_numbers=('NCHW', 'OIHW', 'NCHW'))
    return y + b[None, :, None, None]


def _convT2(x, w, b):
    y = jax.lax.conv_general_dilated(
        x, w, (1, 1), ((1, 2), (1, 2)), lhs_dilation=(2, 2),
        dimension_numbers=('NCHW', 'OIHW', 'NCHW'))
    return y + b[None, :, None, None]


def _vq_kernel(flat_ref, cb_ref, cbsq_ref, idx_ref, acc_ref):
    flat = flat_ref[...]
    cb = cb_ref[...]
    scores = jax.lax.dot_general(
        flat, cb, (((1,), (1,)), ((), ())), preferred_element_type=jnp.float32)
    d2 = cbsq_ref[0, :][None, :] - 2.0 * scores
    m = jnp.min(d2, axis=1)
    iota = jax.lax.broadcasted_iota(jnp.int32, d2.shape, 1)
    idx = jnp.min(jnp.where(d2 <= m[:, None], iota, 512), axis=1)
    idx_ref[0] = idx.reshape(8, idx.shape[0] // 8)
    partial = jnp.sum(m) + jnp.sum(flat * flat)

    @pl.when(pl.program_id(0) == 0)
    def _init():
        acc_ref[...] = jnp.zeros_like(acc_ref)

    acc_ref[...] += partial


def _vq_argmin(flat, codebook):
    n, d = flat.shape
    k = codebook.shape[0]
    tile = 1024
    grid = n // tile
    cbsq = jnp.broadcast_to(jnp.sum(codebook * codebook, axis=1)[None, :], (8, k))
    idx, acc = pl.pallas_call(
        _vq_kernel,
        grid=(grid,),
        in_specs=[
            pl.BlockSpec((tile, d), lambda i: (i, 0)),
            pl.BlockSpec((k, d), lambda i: (0, 0)),
            pl.BlockSpec((8, k), lambda i: (0, 0)),
        ],
        out_specs=[
            pl.BlockSpec((1, 8, tile // 8), lambda i: (i, 0, 0)),
            pl.BlockSpec((8, 128), lambda i: (0, 0)),
        ],
        out_shape=[
            jax.ShapeDtypeStruct((grid, 8, tile // 8), jnp.int32),
            jax.ShapeDtypeStruct((8, 128), jnp.float32),
        ],
    )(flat, codebook, cbsq)
    return idx.reshape(n), acc[0, 0]


def _sc_gather_rows(table, idx):
    b = idx.shape[0]
    d = 128
    table128 = jnp.pad(table, ((0, 0), (0, d - table.shape[1])))
    nw = 32
    chunks = 7
    depth = 4
    b_per_w = b // nw
    b_chunk = b_per_w // chunks
    mesh = plsc.VectorSubcoreMesh(core_axis_name="c", subcore_axis_name="s")

    @functools.partial(
        pl.kernel, mesh=mesh,
        out_type=jax.ShapeDtypeStruct((b, d), jnp.float32),
        scratch_types=[
            pltpu.VMEM((b_per_w,), jnp.int32),
            [pltpu.VMEM((b_chunk, d), jnp.float32) for _ in range(depth)],
            [pltpu.SemaphoreType.DMA for _ in range(depth)],
            [pltpu.SemaphoreType.DMA for _ in range(depth)],
        ],
    )
    def k(table_hbm, idx_hbm, out_hbm, idx_v, rows_v, gsems, wsems):
        wid = jax.lax.axis_index("s") * 2 + jax.lax.axis_index("c")
        base = wid * b_per_w
        pltpu.sync_copy(idx_hbm.at[pl.ds(base, b_per_w)], idx_v)

        def gather(g):
            buf = g % depth
            return pltpu.async_copy(
                table_hbm.at[idx_v.at[pl.ds(g * b_chunk, b_chunk)]],
                rows_v[buf], gsems[buf])

        def writeback(g):
            buf = g % depth
            return pltpu.async_copy(
                rows_v[buf], out_hbm.at[pl.ds(base + g * b_chunk, b_chunk)],
                wsems[buf])

        ghandles = [None] * chunks
        whandles = [None] * chunks
        for g in range(depth):
            ghandles[g] = gather(g)
        for g in range(chunks):
            ghandles[g].wait()
            whandles[g] = writeback(g)
            if g + depth < chunks:
                whandles[g].wait()
                ghandles[g + depth] = gather(g + depth)
        for g in range(max(0, chunks - depth), chunks):
            whandles[g].wait()

    return k(table128, idx)


_W224 = 224
_NPIX = 224 * 224
_PAD = 256
_PADDED = _PAD + _NPIX + 256


def _dec3_loss_kernel(h_ref, x_ref, w_ref, b_ref, acc_ref, scratch_ref):
    @pl.when(pl.program_id(0) == 0)
    def _init():
        scratch_ref[...] = jnp.zeros_like(scratch_ref)
        acc_ref[...] = jnp.zeros_like(acc_ref)

    scratch_ref[:, _PAD:_PAD + _NPIX] = h_ref[0].astype(jnp.bfloat16)
    t = jax.lax.dot_general(
        w_ref[...], scratch_ref[...], (((1,), (0,)), ((), ())),
        preferred_element_type=jnp.float32)
    xcoord = jax.lax.broadcasted_iota(jnp.int32, (1, _NPIX), 1) % _W224
    mneg = xcoord > 0
    mpos = xcoord < _W224 - 1
    r = jnp.zeros((8, _NPIX), jnp.float32)
    for ky in range(3):
        for kx in range(3):
            tap = ky * 3 + kx
            s = (ky - 1) * _W224 + (kx - 1)
            sl = t[8 * tap:8 * tap + 8, _PAD + s:_PAD + s + _NPIX]
            if kx == 0:
                sl = jnp.where(mneg, sl, 0.0)
            elif kx == 2:
                sl = jnp.where(mpos, sl, 0.0)
            r = r + sl
    diff = r[0:3] + b_ref[0:3, 0:1] - x_ref[0]
    acc_ref[...] += jnp.sum(diff * diff)


def _dec3_recon_loss(h, x, w, b):
    n = h.shape[0]
    hf = h.reshape(n, 32, _NPIX)
    xf = x.reshape(n, 3, _NPIX)
    wt = jnp.transpose(w, (2, 3, 0, 1))
    wt = jnp.pad(wt, ((0, 0), (0, 0), (0, 5), (0, 0)))
    w2 = wt.reshape(72, 32).astype(jnp.bfloat16)
    bp = jnp.broadcast_to(jnp.pad(b, (0, 5))[:, None], (8, 128))
    acc = pl.pallas_call(
        _dec3_loss_kernel,
        grid=(n,),
        in_specs=[
            pl.BlockSpec((1, 32, _NPIX), lambda i: (i, 0, 0)),
            pl.BlockSpec((1, 3, _NPIX), lambda i: (i, 0, 0)),
            pl.BlockSpec((72, 32), lambda i: (0, 0)),
            pl.BlockSpec((8, 128), lambda i: (0, 0)),
        ],
        out_specs=pl.BlockSpec((8, 128), lambda i: (0, 0)),
        out_shape=jax.ShapeDtypeStruct((8, 128), jnp.float32),
        scratch_shapes=[pltpu.VMEM((32, _PADDED), jnp.bfloat16)],
    )(hf, xf, w2, bp)
    return acc[0, 0] / x.size


def kernel(x, flag, enc_w1, enc_b1, enc_w2, enc_b2, enc_w3, enc_b3, codebook,
           dec_w1, dec_b1, dec_w2, dec_b2, dec_w3, dec_b3):
    commitment_cost = 0.25
    z = jax.nn.relu(_conv(x, enc_w1, enc_b1, 2, 1))
    z = jax.nn.relu(_conv(z, enc_w2, enc_b2, 2, 1))
    z = _conv(z, enc_w3, enc_b3, 1, 0)
    zp = jnp.transpose(z, (0, 2, 3, 1))
    d = zp.shape[-1]
    flat = zp.reshape(-1, d)
    idx, sqsum = _vq_argmin(flat, codebook)
    vq_loss = (1.0 + commitment_cost) * sqsum / flat.size
    q_flat = _sc_gather_rows(codebook, idx)[:, :d]
    q = q_flat.reshape(zp.shape)
    e = jnp.transpose(q, (0, 3, 1, 2))
    h = jax.nn.relu(_convT2(e, dec_w1, dec_b1))
    h = jax.nn.relu(_convT2(h, dec_w2, dec_b2))
    recon_loss = _dec3_recon_loss(h, x, dec_w3, dec_b3)
    return vq_loss + recon_loss


# speedup vs baseline: 3.0373x; 3.0373x over previous
import functools

import jax
import jax.numpy as jnp
from jax.experimental import pallas as pl
from jax.experimental.pallas import tpu as pltpu
from jax.experimental.pallas import tpu_sc as plsc


def _conv(x, w, b, stride, pad):
    y = jax.lax.conv_general_dilated(
        x, w, (stride, stride), ((pad, pad), (pad, pad)),
        dimension_numbers=('NCHW', 'OIHW', 'NCHW'))
    return y + b[None, :, None, None]


def _convT2(x, w, b):
    y = jax.lax.conv_general_dilated(
        x, w, (1, 1), ((1, 2), (1, 2)), lhs_dilation=(2, 2),
        dimension_numbers=('NCHW', 'OIHW', 'NCHW'))
    return y + b[None, :, None, None]


def _vq_kernel(flat_ref, cb_ref, cbsq_ref, idx_ref, acc_ref):
    flat = flat_ref[...]
    cb = cb_ref[...]
    scores = jax.lax.dot_general(
        flat, cb, (((1,), (1,)), ((), ())), preferred_element_type=jnp.float32)
    d2 = cbsq_ref[0, :][None, :] - 2.0 * scores
    m = jnp.min(d2, axis=1)
    iota = jax.lax.broadcasted_iota(jnp.int32, d2.shape, 1)
    idx = jnp.min(jnp.where(d2 <= m[:, None], iota, 512), axis=1)
    idx_ref[0] = idx.reshape(8, idx.shape[0] // 8)
    partial = jnp.sum(m) + jnp.sum(flat * flat)

    @pl.when(pl.program_id(0) == 0)
    def _init():
        acc_ref[...] = jnp.zeros_like(acc_ref)

    acc_ref[...] += partial


def _vq_argmin(flat, codebook):
    n, d = flat.shape
    k = codebook.shape[0]
    tile = 1024
    grid = n // tile
    cbsq = jnp.broadcast_to(jnp.sum(codebook * codebook, axis=1)[None, :], (8, k))
    idx, acc = pl.pallas_call(
        _vq_kernel,
        grid=(grid,),
        in_specs=[
            pl.BlockSpec((tile, d), lambda i: (i, 0)),
            pl.BlockSpec((k, d), lambda i: (0, 0)),
            pl.BlockSpec((8, k), lambda i: (0, 0)),
        ],
        out_specs=[
            pl.BlockSpec((1, 8, tile // 8), lambda i: (i, 0, 0)),
            pl.BlockSpec((8, 128), lambda i: (0, 0)),
        ],
        out_shape=[
            jax.ShapeDtypeStruct((grid, 8, tile // 8), jnp.int32),
            jax.ShapeDtypeStruct((8, 128), jnp.float32),
        ],
    )(flat, codebook, cbsq)
    return idx.reshape(n), acc[0, 0]


def _sc_gather_rows(table, idx):
    b = idx.shape[0]
    d = 128
    table128 = jnp.pad(table, ((0, 0), (0, d - table.shape[1])))
    nw = 32
    chunks = 7
    depth = 4
    b_per_w = b // nw
    b_chunk = b_per_w // chunks
    mesh = plsc.VectorSubcoreMesh(core_axis_name="c", subcore_axis_name="s")

    @functools.partial(
        pl.kernel, mesh=mesh,
        out_type=jax.ShapeDtypeStruct((b, d), jnp.float32),
        scratch_types=[
            pltpu.VMEM((b_per_w,), jnp.int32),
            pltpu.VMEM_SHARED((512, 128), jnp.float32),
            [pltpu.VMEM((b_chunk, d), jnp.float32) for _ in range(depth)],
            [pltpu.SemaphoreType.DMA for _ in range(depth)],
            [pltpu.SemaphoreType.DMA for _ in range(depth)],
        ],
    )
    def k(table_hbm, idx_hbm, out_hbm, idx_v, table_s, rows_v, gsems, wsems):
        sid = jax.lax.axis_index("s")
        wid = sid * 2 + jax.lax.axis_index("c")
        base = wid * b_per_w

        @pl.when(sid == 0)
        def _stage_table():
            pltpu.sync_copy(table_hbm, table_s)

        pltpu.sync_copy(idx_hbm.at[pl.ds(base, b_per_w)], idx_v)
        plsc.subcore_barrier()

        def gather(g):
            buf = g % depth
            return pltpu.async_copy(
                table_s.at[idx_v.at[pl.ds(g * b_chunk, b_chunk)]],
                rows_v[buf], gsems[buf])

        def writeback(g):
            buf = g % depth
            return pltpu.async_copy(
                rows_v[buf], out_hbm.at[pl.ds(base + g * b_chunk, b_chunk)],
                wsems[buf])

        ghandles = [None] * chunks
        whandles = [None] * chunks
        for g in range(depth):
            ghandles[g] = gather(g)
        for g in range(chunks):
            ghandles[g].wait()
            whandles[g] = writeback(g)
            if g + depth < chunks:
                whandles[g].wait()
                ghandles[g + depth] = gather(g + depth)
        for g in range(max(0, chunks - depth), chunks):
            whandles[g].wait()

    return k(table128, idx)


_W224 = 224
_NPIX = 224 * 224
_PAD = 256
_PADDED = _PAD + _NPIX + 256


def _dec3_loss_kernel(h_ref, x_ref, w_ref, b_ref, acc_ref, scratch_ref):
    @pl.when(pl.program_id(0) == 0)
    def _init():
        scratch_ref[...] = jnp.zeros_like(scratch_ref)
        acc_ref[...] = jnp.zeros_like(acc_ref)

    scratch_ref[:, _PAD:_PAD + _NPIX] = h_ref[0].astype(jnp.bfloat16)
    t = jax.lax.dot_general(
        w_ref[...], scratch_ref[...], (((1,), (0,)), ((), ())),
        preferred_element_type=jnp.float32)
    xcoord = jax.lax.broadcasted_iota(jnp.int32, (1, _NPIX), 1) % _W224
    mneg = xcoord > 0
    mpos = xcoord < _W224 - 1
    r = jnp.zeros((8, _NPIX), jnp.float32)
    for ky in range(3):
        for kx in range(3):
            tap = ky * 3 + kx
            s = (ky - 1) * _W224 + (kx - 1)
            sl = t[8 * tap:8 * tap + 8, _PAD + s:_PAD + s + _NPIX]
            if kx == 0:
                sl = jnp.where(mneg, sl, 0.0)
            elif kx == 2:
                sl = jnp.where(mpos, sl, 0.0)
            r = r + sl
    diff = r[0:3] + b_ref[0:3, 0:1] - x_ref[0]
    acc_ref[...] += jnp.sum(diff * diff)


def _dec3_recon_loss(h, x, w, b):
    n = h.shape[0]
    hf = h.reshape(n, 32, _NPIX)
    xf = x.reshape(n, 3, _NPIX)
    wt = jnp.transpose(w, (2, 3, 0, 1))
    wt = jnp.pad(wt, ((0, 0), (0, 0), (0, 5), (0, 0)))
    w2 = wt.reshape(72, 32).astype(jnp.bfloat16)
    bp = jnp.broadcast_to(jnp.pad(b, (0, 5))[:, None], (8, 128))
    acc = pl.pallas_call(
        _dec3_loss_kernel,
        grid=(n,),
        in_specs=[
            pl.BlockSpec((1, 32, _NPIX), lambda i: (i, 0, 0)),
            pl.BlockSpec((1, 3, _NPIX), lambda i: (i, 0, 0)),
            pl.BlockSpec((72, 32), lambda i: (0, 0)),
            pl.BlockSpec((8, 128), lambda i: (0, 0)),
        ],
        out_specs=pl.BlockSpec((8, 128), lambda i: (0, 0)),
        out_shape=jax.ShapeDtypeStruct((8, 128), jnp.float32),
        scratch_shapes=[pltpu.VMEM((32, _PADDED), jnp.bfloat16)],
    )(hf, xf, w2, bp)
    return acc[0, 0] / x.size


def kernel(x, flag, enc_w1, enc_b1, enc_w2, enc_b2, enc_w3, enc_b3, codebook,
           dec_w1, dec_b1, dec_w2, dec_b2, dec_w3, dec_b3):
    commitment_cost = 0.25
    z = jax.nn.relu(_conv(x, enc_w1, enc_b1, 2, 1))
    z = jax.nn.relu(_conv(z, enc_w2, enc_b2, 2, 1))
    z = _conv(z, enc_w3, enc_b3, 1, 0)
    zp = jnp.transpose(z, (0, 2, 3, 1))
    d = zp.shape[-1]
    flat = zp.reshape(-1, d)
    idx, sqsum = _vq_argmin(flat, codebook)
    vq_loss = (1.0 + commitment_cost) * sqsum / flat.size
    q_flat = _sc_gather_rows(codebook, idx)[:, :d]
    q = q_flat.reshape(zp.shape)
    e = jnp.transpose(q, (0, 3, 1, 2))
    h = jax.nn.relu(_convT2(e, dec_w1, dec_b1))
    h = jax.nn.relu(_convT2(h, dec_w2, dec_b2))
    recon_loss = _dec3_recon_loss(h, x, dec_w3, dec_b3)
    return vq_loss + recon_loss
